# baseline (speedup 1.0000x reference)
import functools

import jax
import jax.numpy as jnp
from jax import lax
from jax.experimental import pallas as pl
from jax.experimental.pallas import tpu as pltpu
from jax.experimental.pallas import tpu_sc as plsc

_D_MODEL = 4096
_N_EXPERT = 64
_TOP_K = 8
_BLOCK_T = 1024
_NC, _NS, _L = 2, 16, 16
_NW = _NC * _NS

_GATHER_DNUMS = lax.GatherDimensionNumbers(
    offset_dims=(), collapsed_slice_dims=(0,), start_index_map=(0,))


def _probs_block(x_ref, w_ref, probs_ref):
    x = x_ref[...]
    w = w_ref[...]
    logits_t = jax.lax.dot_general(
        w, x, (((1,), (1,)), ((), ())),
        preferred_element_type=jnp.float32)
    m = jnp.max(logits_t, axis=0, keepdims=True)
    e = jnp.exp(logits_t - m)
    probs_t = e / jnp.sum(e, axis=0, keepdims=True)
    probs_ref[...] = probs_t.T


def _tc_probs(x, W_gate):
    n_tokens, d_model = x.shape
    n_expert = W_gate.shape[0]
    return pl.pallas_call(
        _probs_block,
        grid=(n_tokens // _BLOCK_T,),
        in_specs=[
            pl.BlockSpec((_BLOCK_T, d_model), lambda i: (i, 0)),
            pl.BlockSpec((n_expert, d_model), lambda i: (0, 0)),
        ],
        out_specs=pl.BlockSpec((_BLOCK_T, n_expert), lambda i: (i, 0)),
        out_shape=jax.ShapeDtypeStruct((n_tokens, n_expert), jnp.float32),
        compiler_params=pltpu.CompilerParams(
            dimension_semantics=("parallel",)),
    )(x, W_gate)


def _shuf(v, idx):
    return lax.gather(v, idx[:, None], _GATHER_DNUMS, (1,),
                      mode=lax.GatherScatterMode.PROMISE_IN_BOUNDS)


def _bfly(v, op, lane):
    for k in (8, 4, 2, 1):
        v = op(v, _shuf(v, lane ^ k))
    return v


def _make_sc_topk(n_tokens):
    rows = n_tokens // _NW
    mesh = plsc.VectorSubcoreMesh(core_axis_name="c", subcore_axis_name="s")

    @functools.partial(
        pl.kernel,
        out_type=[
            jax.ShapeDtypeStruct((n_tokens * _L,), jnp.float32),
            jax.ShapeDtypeStruct((n_tokens * _L,), jnp.int32),
        ],
        mesh=mesh,
        scratch_types=[
            pltpu.VMEM((rows * _N_EXPERT,), jnp.float32),
            pltpu.VMEM((rows * _L,), jnp.float32),
            pltpu.VMEM((rows * _L,), jnp.int32),
        ],
    )
    def sc_topk(probs_hbm, tp_hbm, ti_hbm, probs_v, tp_v, ti_v):
        wid = lax.axis_index("s") * _NC + lax.axis_index("c")
        base = wid * rows
        pltpu.sync_copy(probs_hbm.at[pl.ds(base * _N_EXPERT,
                                           rows * _N_EXPERT)], probs_v)
        lane = lax.iota(jnp.int32, _L)
        glane = [lane + g * _L for g in range(4)]

        def body(t, carry):
            v = [probs_v[pl.ds(t * _N_EXPERT + g * _L, _L)]
                 for g in range(4)]
            tpvec = jnp.zeros((_L,), jnp.float32)
            tivec = jnp.zeros((_L,), jnp.int32)
            for k in range(_TOP_K):
                m = jnp.maximum(jnp.maximum(v[0], v[1]),
                                jnp.maximum(v[2], v[3]))
                m = _bfly(m, jnp.maximum, lane)
                cand = jnp.minimum(
                    jnp.minimum(
                        jnp.where(v[0] == m, glane[0], _N_EXPERT),
                        jnp.where(v[1] == m, glane[1], _N_EXPERT)),
                    jnp.minimum(
                        jnp.where(v[2] == m, glane[2], _N_EXPERT),
                        jnp.where(v[3] == m, glane[3], _N_EXPERT)))
                idx = _bfly(cand, jnp.minimum, lane)
                tpvec = jnp.where(lane == k, m, tpvec)
                tivec = jnp.where(lane == k, idx, tivec)
                v = [jnp.where(glane[g] == idx, -1.0, v[g])
                     for g in range(4)]
            ssum = _bfly(jnp.where(lane < _TOP_K, tpvec, 0.0),
                         jnp.add, lane)
            tp_v[pl.ds(t * _L, _L)] = tpvec / ssum
            ti_v[pl.ds(t * _L, _L)] = tivec
            return carry

        lax.fori_loop(0, rows, body, 0)
        pltpu.sync_copy(tp_v, tp_hbm.at[pl.ds(base * _L, rows * _L)])
        pltpu.sync_copy(ti_v, ti_hbm.at[pl.ds(base * _L, rows * _L)])

    return sc_topk


def kernel(x, W_gate):
    n_tokens = x.shape[0]
    probs = _tc_probs(x, W_gate)
    tp16, ti16 = _make_sc_topk(n_tokens)(probs.reshape(-1))
    tp = tp16.reshape(n_tokens, _L)[:, :_TOP_K]
    ti = ti16.reshape(n_tokens, _L)[:, :_TOP_K]
    return (tp, ti, probs)


# speedup vs baseline: 2.0973x; 2.0973x over previous
import functools

import jax
import jax.numpy as jnp
from jax.experimental import pallas as pl
from jax.experimental.pallas import tpu as pltpu

_D_MODEL = 4096
_N_EXPERT = 64
_TOP_K = 8
_BLOCK_T = 1024


def _router_block(x_ref, w_ref, probs_ref, tp_ref, ti_ref):
    x = x_ref[...]
    w = w_ref[...]
    logits_t = jax.lax.dot_general(
        w, x, (((1,), (1,)), ((), ())),
        preferred_element_type=jnp.float32)

    m = jnp.max(logits_t, axis=0, keepdims=True)
    e = jnp.exp(logits_t - m)
    probs_t = e / jnp.sum(e, axis=0, keepdims=True)
    probs_ref[...] = probs_t.T

    row = jax.lax.broadcasted_iota(jnp.int32, probs_t.shape, 0)
    work = probs_t
    tps = []
    tis = []
    for _ in range(_TOP_K):
        mx = jnp.max(work, axis=0, keepdims=True)
        idx = jnp.min(jnp.where(work == mx, row, _N_EXPERT),
                      axis=0, keepdims=True)
        tps.append(mx)
        tis.append(idx)
        work = jnp.where(row == idx, -1.0, work)

    tp_t = jnp.concatenate(tps, axis=0)
    ti_t = jnp.concatenate(tis, axis=0)
    tp_t = tp_t / jnp.sum(tp_t, axis=0, keepdims=True)
    tp_ref[...] = tp_t.T
    ti_ref[...] = ti_t.T


def kernel(x, W_gate):
    n_tokens, d_model = x.shape
    n_expert = W_gate.shape[0]
    grid = (n_tokens // _BLOCK_T,)
    probs, tp, ti = pl.pallas_call(
        _router_block,
        grid=grid,
        in_specs=[
            pl.BlockSpec((_BLOCK_T, d_model), lambda i: (i, 0)),
            pl.BlockSpec((n_expert, d_model), lambda i: (0, 0)),
        ],
        out_specs=[
            pl.BlockSpec((_BLOCK_T, n_expert), lambda i: (i, 0)),
            pl.BlockSpec((_BLOCK_T, _TOP_K), lambda i: (i, 0)),
            pl.BlockSpec((_BLOCK_T, _TOP_K), lambda i: (i, 0)),
        ],
        out_shape=[
            jax.ShapeDtypeStruct((n_tokens, n_expert), jnp.float32),
            jax.ShapeDtypeStruct((n_tokens, _TOP_K), jnp.float32),
            jax.ShapeDtypeStruct((n_tokens, _TOP_K), jnp.int32),
        ],
        compiler_params=pltpu.CompilerParams(
            dimension_semantics=("parallel",)),
    )(x, W_gate)
    return (tp, ti, probs)
